# baseline (speedup 1.0000x reference)
import jax
import jax.numpy as jnp
from jax import lax
from jax.experimental import pallas as pl
from jax.experimental.pallas import tpu as pltpu
from jax.experimental.pallas import tpu_sc as plsc

_B = 16384
_EMB = 32
_NC = 2
_NS = 16
_NW = _NC * _NS
_BPW = _B // _NW


def _sc_gather_body(users_hbm, items_hbm, uemb_hbm, iemb_hbm, ulin_hbm, ilin_hbm,
                    ue_out, ie_out, ul_out, il_out,
                    uidx_v, iidx_v, ue_v, ie_v, ul_v, il_v, sem):
    wid = lax.axis_index("s") * _NC + lax.axis_index("c")
    base = wid * _BPW
    pltpu.sync_copy(users_hbm.at[pl.ds(base, _BPW)], uidx_v)
    pltpu.sync_copy(items_hbm.at[pl.ds(base, _BPW)], iidx_v)
    c1 = pltpu.async_copy(uemb_hbm.at[uidx_v], ue_v, sem)
    c2 = pltpu.async_copy(iemb_hbm.at[iidx_v], ie_v, sem)
    c3 = pltpu.async_copy(ulin_hbm.at[uidx_v], ul_v, sem)
    c4 = pltpu.async_copy(ilin_hbm.at[iidx_v], il_v, sem)
    c1.wait()
    c2.wait()
    c3.wait()
    c4.wait()
    pltpu.sync_copy(ue_v, ue_out.at[pl.ds(base, _BPW)])
    pltpu.sync_copy(ie_v, ie_out.at[pl.ds(base, _BPW)])
    pltpu.sync_copy(ul_v, ul_out.at[pl.ds(base, _BPW)])
    pltpu.sync_copy(il_v, il_out.at[pl.ds(base, _BPW)])


def _sc_gather(users, items, uemb, iemb, ulin, ilin):
    mesh = plsc.VectorSubcoreMesh(core_axis_name="c", subcore_axis_name="s")
    f = pl.kernel(
        _sc_gather_body,
        mesh=mesh,
        compiler_params=pltpu.CompilerParams(use_tc_tiling_on_sc=False),
        out_type=[
            jax.ShapeDtypeStruct((_B, _EMB), jnp.float32),
            jax.ShapeDtypeStruct((_B, _EMB), jnp.float32),
            jax.ShapeDtypeStruct((_B,), jnp.float32),
            jax.ShapeDtypeStruct((_B,), jnp.float32),
        ],
        scratch_types=[
            pltpu.VMEM((_BPW,), jnp.int32),
            pltpu.VMEM((_BPW,), jnp.int32),
            pltpu.VMEM((_BPW, _EMB), jnp.float32),
            pltpu.VMEM((_BPW, _EMB), jnp.float32),
            pltpu.VMEM((_BPW,), jnp.float32),
            pltpu.VMEM((_BPW,), jnp.float32),
            pltpu.SemaphoreType.DMA,
        ],
    )
    return f(users, items, uemb, iemb, ulin, ilin)


_BLK = 2048


def _tc_body(ue_ref, ie_ref, ul_ref, il_ref,
             w1u_ref, w1i_ref, b1_ref, w2_ref, b2_ref, w3_ref, b3_ref,
             out_ref, aux_ref):
    ue = ue_ref[...]
    ie = ie_ref[...]
    h = jnp.dot(ue, w1u_ref[...], preferred_element_type=jnp.float32)
    h = h + jnp.dot(ie, w1i_ref[...], preferred_element_type=jnp.float32)
    h = jax.nn.relu(h + b1_ref[...])
    h = jax.nn.relu(jnp.dot(h, w2_ref[...], preferred_element_type=jnp.float32)
                    + b2_ref[...])
    mlp = jnp.sum(h * w3_ref[...], axis=1, keepdims=True) + b3_ref[...]
    fm = jnp.sum(ue * ie, axis=1, keepdims=True)
    out_ref[...] = jax.nn.sigmoid(ul_ref[...] + il_ref[...] + fm + mlp)

    @pl.when(pl.program_id(0) == 0)
    def _():
        aux_ref[...] = jnp.zeros_like(aux_ref)

    aux_ref[...] += jnp.sum(mlp * mlp)


def _tc_forward(ue, ie, ul, il, w1u, w1i, b1, w2, b2, w3t, b3):
    grid = (_B // _BLK,)
    full = lambda shape: pl.BlockSpec(shape, lambda i: (0, 0))
    out, aux = pl.pallas_call(
        _tc_body,
        grid=grid,
        in_specs=[
            pl.BlockSpec((_BLK, _EMB), lambda i: (i, 0)),
            pl.BlockSpec((_BLK, _EMB), lambda i: (i, 0)),
            pl.BlockSpec((_BLK, 1), lambda i: (i, 0)),
            pl.BlockSpec((_BLK, 1), lambda i: (i, 0)),
            full(w1u.shape),
            full(w1i.shape),
            full(b1.shape),
            full(w2.shape),
            full(b2.shape),
            full(w3t.shape),
            full(b3.shape),
        ],
        out_specs=[
            pl.BlockSpec((_BLK, 1), lambda i: (i, 0)),
            pl.BlockSpec((1, 1), lambda i: (0, 0)),
        ],
        out_shape=[
            jax.ShapeDtypeStruct((_B, 1), jnp.float32),
            jax.ShapeDtypeStruct((1, 1), jnp.float32),
        ],
    )(ue, ie, ul, il, w1u, w1i, b1, w2, b2, w3t, b3)
    return out, aux


def kernel(users, items, user_linear, item_linear, user_emb, item_emb,
           W1, b1, W2, b2, W3, b3):
    users = users.reshape(-1)
    items = items.reshape(-1)
    ue, ie, ul, il = _sc_gather(users, items, user_emb, item_emb,
                                user_linear.reshape(-1), item_linear.reshape(-1))
    out, aux_sum = _tc_forward(
        ue, ie, ul.reshape(_B, 1), il.reshape(_B, 1),
        W1[:_EMB], W1[_EMB:], b1.reshape(1, -1),
        W2, b2.reshape(1, -1), W3.reshape(1, -1), b3.reshape(1, 1),
    )
    aux = 0.1 * aux_sum[0, 0] / _B
    return out, aux
